# baseline (speedup 1.0000x reference)
import functools

import jax
import jax.numpy as jnp
from jax import lax
from jax.experimental import pallas as pl
from jax.experimental.pallas import tpu as pltpu
from jax.experimental.pallas import tpu_sc as plsc

D_MODEL = 64
SCALE = 8.0

NC = 2
NS = 16
NW = NC * NS
LANES = 16

SB = 8
BB = 128

VOCAB_PAD = 1000064
NBLK = VOCAB_PAD // 128
ABLK = 245


@jax.jit
def _format_table(table_t):
    mesh = plsc.VectorSubcoreMesh(
        core_axis_name="c", subcore_axis_name="s", num_cores=NC, num_subcores=NS
    )

    @functools.partial(
        pl.kernel,
        out_type=jax.ShapeDtypeStruct((VOCAB_PAD * D_MODEL,), jnp.float32),
        mesh=mesh,
        scratch_types=[
            pltpu.VMEM((2, D_MODEL, 128), jnp.float32),
            pltpu.VMEM((D_MODEL * 128,), jnp.float32),
            pltpu.VMEM((D_MODEL * 128,), jnp.float32),
            pltpu.SemaphoreType.DMA((2,)),
            pltpu.SemaphoreType.DMA((2,)),
        ],
        compiler_params=pltpu.CompilerParams(
            use_tc_tiling_on_sc=True,
            needs_layout_passes=False,
            disable_bounds_checks=True,
        ),
    )
    def body(tt_hbm, scr_hbm, stage, obuf_0, obuf_1, sem_i, sem_o):
        obufs = [obuf_0, obuf_1]
        wid = lax.axis_index("s") * NC + lax.axis_index("c")
        base = (NBLK * wid) // NW
        iota = lax.iota(jnp.int32, LANES)
        rot = [(iota + dd) & 15 for dd in range(LANES)]
        q64 = [iota * D_MODEL + rot[dd] for dd in range(LANES)]

        def start_in(i, slot):
            v0 = pl.multiple_of((base + i) * 128, 128)
            return pltpu.async_copy(
                tt_hbm.at[:, pl.ds(v0, 128)], stage.at[slot], sem_i.at[slot]
            )

        def start_out(i, slot):
            r0 = pl.multiple_of((base + i) * (128 * D_MODEL), 128 * D_MODEL)
            return pltpu.async_copy(
                obufs[slot],
                scr_hbm.at[pl.ds(r0, 128 * D_MODEL)],
                sem_o.at[slot],
            )

        def transpose_block(slot):
            def one_tile(t, _):
                jd = t // 2
                gq = (t % 2) * 4
                vvs = [iota + (gq + u) * 16 for u in range(4)]
                woffs = [(gq + u) * (16 * D_MODEL) + jd * 16 for u in range(4)]
                for dd in range(LANES):
                    dv = rot[dd] + jd * 16
                    vecs = [
                        plsc.load_gather(stage.at[slot], [dv, vvs[u]])
                        for u in range(4)
                    ]
                    for u in range(4):
                        plsc.store_scatter(
                            obufs[slot], [q64[dd] + woffs[u]], vecs[u] * SCALE
                        )
                return ()

            lax.fori_loop(0, 8, one_tile, ())

        start_in(0, 0)
        start_in(1, 1)

        def step(i2, _):
            for b in range(2):
                i = i2 * 2 + b
                pltpu.make_async_copy(
                    tt_hbm.at[:, pl.ds(0, 128)], stage.at[b], sem_i.at[b]
                ).wait()

                @pl.when(i2 >= 1)
                def _():
                    pltpu.make_async_copy(
                        obufs[b],
                        scr_hbm.at[pl.ds(0, 128 * D_MODEL)],
                        sem_o.at[b],
                    ).wait()

                transpose_block(b)
                start_out(i, b)
                if b == 0:
                    start_in(i + 2, 0)
                else:
                    @pl.when(i2 < (ABLK - 3) // 2)
                    def _():
                        start_in(i + 2, 1)
            return ()

        lax.fori_loop(0, (ABLK - 1) // 2, step, ())
        i_last = ABLK - 1
        pltpu.make_async_copy(
            tt_hbm.at[:, pl.ds(0, 128)], stage.at[0], sem_i.at[0]
        ).wait()
        pltpu.make_async_copy(
            obufs[0], scr_hbm.at[pl.ds(0, 128 * D_MODEL)], sem_o.at[0]
        ).wait()
        transpose_block(0)
        start_out(i_last, 0)
        pltpu.make_async_copy(
            obufs[1], scr_hbm.at[pl.ds(0, 128 * D_MODEL)], sem_o.at[1]
        ).wait()
        pltpu.make_async_copy(
            obufs[0], scr_hbm.at[pl.ds(0, 128 * D_MODEL)], sem_o.at[0]
        ).wait()

    return body(table_t)


@jax.jit
def _embed_lookup(x4, t2):
    n_sblk, n_bblk = x4.shape[0], x4.shape[1]
    per_w = (n_sblk * n_bblk) // NW
    mesh = plsc.VectorSubcoreMesh(
        core_axis_name="c", subcore_axis_name="s", num_cores=NC, num_subcores=NS
    )

    @functools.partial(
        pl.kernel,
        out_type=jax.ShapeDtypeStruct(
            (n_sblk * SB, D_MODEL // 8, n_bblk, 8 * BB), jnp.float32
        ),
        mesh=mesh,
        scratch_types=[
            pltpu.VMEM((SB, BB), jnp.int32),
            pltpu.VMEM((4, BB, D_MODEL), jnp.float32),
            pltpu.VMEM((D_MODEL * BB,), jnp.float32),
            pltpu.VMEM((D_MODEL * BB,), jnp.float32),
            pltpu.SemaphoreType.DMA((4,)),
            pltpu.SemaphoreType.DMA((2,)),
        ],
        compiler_params=pltpu.CompilerParams(
            use_tc_tiling_on_sc=False, needs_layout_passes=False
        ),
    )
    def body(x_hbm, t_hbm, out_hbm, idx_v, rows_v, obuf_0, obuf_1, sem_g, sem_o):
        obufs = [obuf_0, obuf_1]
        wid = lax.axis_index("s") * NC + lax.axis_index("c")
        iota = lax.iota(jnp.int32, LANES)
        rot = [(iota + dd) & 15 for dd in range(LANES)]
        p128 = [rot[dd] * BB + iota for dd in range(LANES)]

        def do_item(i, _):
            it = wid * per_w + i
            st = it // n_bblk
            bt = it % n_bblk
            pltpu.sync_copy(x_hbm.at[st, bt], idx_v)

            def start_gather(k):
                return pltpu.async_copy(
                    t_hbm.at[idx_v.at[k]], rows_v.at[k % 4], sem_g.at[k % 4]
                )

            def start_out(k):
                s = st * SB + k
                return [
                    pltpu.async_copy(
                        obufs[k % 2].at[pl.ds(dt * (8 * BB), 8 * BB)],
                        out_hbm.at[s, dt, bt],
                        sem_o.at[k % 2],
                    )
                    for dt in range(D_MODEL // 8)
                ]

            def transpose_rows(jr, jo):
                def one_tile(gb, _):
                    bb = iota + gb * 16
                    woffs = [jd * (16 * BB) + gb * 16 for jd in range(4)]
                    for dd in range(LANES):
                        vecs = [
                            plsc.load_gather(
                                rows_v.at[jr], [bb, rot[dd] + jd * 16]
                            )
                            for jd in range(4)
                        ]
                        for jd in range(4):
                            plsc.store_scatter(
                                obufs[jo], [p128[dd] + woffs[jd]], vecs[jd]
                            )
                    return ()

                lax.fori_loop(0, 8, one_tile, ())

            gathers = [start_gather(0), start_gather(1), start_gather(2)]
            outs = []
            for k in range(SB):
                if k + 3 < SB:
                    gathers.append(start_gather(k + 3))
                gathers[k].wait()
                if k >= 2:
                    for c in outs[k - 2]:
                        c.wait()
                transpose_rows(k % 4, k % 2)
                outs.append(start_out(k))
            for c in outs[SB - 2]:
                c.wait()
            for c in outs[SB - 1]:
                c.wait()
            return ()

        lax.fori_loop(0, per_w, do_item, ())

    return body(x4, t2)


def kernel(x, table):
    b, s = x.shape
    n_sblk, n_bblk = s // SB, b // BB
    x4 = (
        x.T.astype(jnp.int32)
        .reshape(n_sblk, SB, n_bblk, BB)
        .transpose(0, 2, 1, 3)
    )
    scr = _format_table(table.T)
    t2 = scr.reshape(VOCAB_PAD, D_MODEL)
    o4 = _embed_lookup(x4, t2)
    o5 = o4.reshape(s, D_MODEL // 8, n_bblk, 8, BB)
    return o5.transpose(2, 4, 0, 1, 3).reshape(b, s, D_MODEL)


# speedup vs baseline: 1.0501x; 1.0501x over previous
import functools

import jax
import jax.numpy as jnp
from jax import lax
from jax.experimental import pallas as pl
from jax.experimental.pallas import tpu as pltpu
from jax.experimental.pallas import tpu_sc as plsc

D_MODEL = 64
SCALE = 8.0

NC = 2
NS = 16
NW = NC * NS
LANES = 16

SB = 8
BB = 128

VOCAB_PAD = 1000064
NBLK = VOCAB_PAD // 128
ABLK = 245


@jax.jit
def _format_table(table_t):
    mesh = plsc.VectorSubcoreMesh(
        core_axis_name="c", subcore_axis_name="s", num_cores=NC, num_subcores=NS
    )

    @functools.partial(
        pl.kernel,
        out_type=jax.ShapeDtypeStruct((VOCAB_PAD * D_MODEL,), jnp.float32),
        mesh=mesh,
        scratch_types=[
            pltpu.VMEM((2, D_MODEL, 128), jnp.float32),
            pltpu.VMEM((D_MODEL * 128,), jnp.float32),
            pltpu.VMEM((D_MODEL * 128,), jnp.float32),
            pltpu.SemaphoreType.DMA((2,)),
            pltpu.SemaphoreType.DMA((2,)),
        ],
        compiler_params=pltpu.CompilerParams(
            use_tc_tiling_on_sc=True,
            needs_layout_passes=False,
            disable_bounds_checks=True,
        ),
    )
    def body(tt_hbm, scr_hbm, stage, obuf_0, obuf_1, sem_i, sem_o):
        obufs = [obuf_0, obuf_1]
        wid = lax.axis_index("s") * NC + lax.axis_index("c")
        base = (NBLK * wid) // NW
        iota = lax.iota(jnp.int32, LANES)
        rot = [(iota + dd) & 15 for dd in range(LANES)]
        q64 = [iota * D_MODEL + rot[dd] for dd in range(LANES)]

        def start_in(i, slot):
            v0 = pl.multiple_of((base + i) * 128, 128)
            return pltpu.async_copy(
                tt_hbm.at[:, pl.ds(v0, 128)], stage.at[slot], sem_i.at[slot]
            )

        def start_out(i, slot):
            r0 = pl.multiple_of((base + i) * (128 * D_MODEL), 128 * D_MODEL)
            return pltpu.async_copy(
                obufs[slot],
                scr_hbm.at[pl.ds(r0, 128 * D_MODEL)],
                sem_o.at[slot],
            )

        def transpose_block(slot):
            def one_tile(t, _):
                jd = t // 2
                gq = (t % 2) * 4
                vvs = [iota + (gq + u) * 16 for u in range(4)]
                woffs = [(gq + u) * (16 * D_MODEL) + jd * 16 for u in range(4)]
                for dd in range(LANES):
                    dv = rot[dd] + jd * 16
                    vecs = [
                        plsc.load_gather(stage.at[slot], [dv, vvs[u]])
                        for u in range(4)
                    ]
                    for u in range(4):
                        plsc.store_scatter(
                            obufs[slot], [q64[dd] + woffs[u]], vecs[u] * SCALE
                        )
                return ()

            lax.fori_loop(0, 8, one_tile, ())

        start_in(0, 0)
        start_in(1, 1)

        def step(i2, _):
            for b in range(2):
                i = i2 * 2 + b
                pltpu.make_async_copy(
                    tt_hbm.at[:, pl.ds(0, 128)], stage.at[b], sem_i.at[b]
                ).wait()

                @pl.when(i2 >= 1)
                def _():
                    pltpu.make_async_copy(
                        obufs[b],
                        scr_hbm.at[pl.ds(0, 128 * D_MODEL)],
                        sem_o.at[b],
                    ).wait()

                transpose_block(b)
                start_out(i, b)
                if b == 0:
                    start_in(i + 2, 0)
                else:
                    @pl.when(i2 < (ABLK - 3) // 2)
                    def _():
                        start_in(i + 2, 1)
            return ()

        lax.fori_loop(0, (ABLK - 1) // 2, step, ())
        i_last = ABLK - 1
        pltpu.make_async_copy(
            tt_hbm.at[:, pl.ds(0, 128)], stage.at[0], sem_i.at[0]
        ).wait()
        pltpu.make_async_copy(
            obufs[0], scr_hbm.at[pl.ds(0, 128 * D_MODEL)], sem_o.at[0]
        ).wait()
        transpose_block(0)
        start_out(i_last, 0)
        pltpu.make_async_copy(
            obufs[1], scr_hbm.at[pl.ds(0, 128 * D_MODEL)], sem_o.at[1]
        ).wait()
        pltpu.make_async_copy(
            obufs[0], scr_hbm.at[pl.ds(0, 128 * D_MODEL)], sem_o.at[0]
        ).wait()

    return body(table_t)


@jax.jit
def _embed_lookup(x4, t2):
    n_sblk, n_bblk = x4.shape[0], x4.shape[1]
    per_w = (n_sblk * n_bblk) // NW
    mesh = plsc.VectorSubcoreMesh(
        core_axis_name="c", subcore_axis_name="s", num_cores=NC, num_subcores=NS
    )

    @functools.partial(
        pl.kernel,
        out_type=jax.ShapeDtypeStruct(
            (n_sblk * SB, D_MODEL // 8, n_bblk, 8 * BB), jnp.float32
        ),
        mesh=mesh,
        scratch_types=[
            pltpu.VMEM((2, SB, BB), jnp.int32),
            pltpu.VMEM((4, BB, D_MODEL), jnp.float32),
            pltpu.VMEM((D_MODEL * BB,), jnp.float32),
            pltpu.VMEM((D_MODEL * BB,), jnp.float32),
            pltpu.SemaphoreType.DMA((4,)),
            pltpu.SemaphoreType.DMA((2,)),
            pltpu.SemaphoreType.DMA((2,)),
        ],
        compiler_params=pltpu.CompilerParams(
            use_tc_tiling_on_sc=False, needs_layout_passes=False
        ),
    )
    def body(
        x_hbm, t_hbm, out_hbm, idx_v, rows_v, obuf_0, obuf_1,
        sem_g, sem_o, sem_x,
    ):
        obufs = [obuf_0, obuf_1]
        wid = lax.axis_index("s") * NC + lax.axis_index("c")
        iota = lax.iota(jnp.int32, LANES)
        rot = [(iota + dd) & 15 for dd in range(LANES)]
        p128 = [rot[dd] * BB + iota for dd in range(LANES)]

        def coords(i_local):
            it = wid * per_w + i_local
            return it // n_bblk, it % n_bblk

        def fire_idx(i_local, xs):
            st, bt = coords(i_local)
            pltpu.async_copy(x_hbm.at[st, bt], idx_v.at[xs], sem_x.at[xs])

        def wait_idx(xs):
            pltpu.make_async_copy(
                x_hbm.at[0, 0], idx_v.at[xs], sem_x.at[xs]
            ).wait()

        def fire_gather(k, gs, xs):
            pltpu.async_copy(
                t_hbm.at[idx_v.at[xs, k]], rows_v.at[gs], sem_g.at[gs]
            )

        def wait_gather(gs):
            pltpu.make_async_copy(
                t_hbm.at[idx_v.at[0, 0]], rows_v.at[gs], sem_g.at[gs]
            ).wait()

        def fire_out(i_local, k):
            st, bt = coords(i_local)
            s = st * SB + k
            for dt in range(D_MODEL // 8):
                pltpu.async_copy(
                    obufs[k % 2].at[pl.ds(dt * (8 * BB), 8 * BB)],
                    out_hbm.at[s, dt, bt],
                    sem_o.at[k % 2],
                )

        def wait_out(oslot):
            for dt in range(D_MODEL // 8):
                pltpu.make_async_copy(
                    obufs[oslot].at[pl.ds(dt * (8 * BB), 8 * BB)],
                    out_hbm.at[0, 0, 0],
                    sem_o.at[oslot],
                ).wait()

        def transpose_rows(jr, jo):
            def one_tile(gb, _):
                bb = iota + gb * 16
                woffs = [jd * (16 * BB) + gb * 16 for jd in range(4)]
                for dd in range(LANES):
                    vecs = [
                        plsc.load_gather(
                            rows_v.at[jr], [bb, rot[dd] + jd * 16]
                        )
                        for jd in range(4)
                    ]
                    for jd in range(4):
                        plsc.store_scatter(
                            obufs[jo], [p128[dd] + woffs[jd]], vecs[jd]
                        )
                return ()

            lax.fori_loop(0, 8, one_tile, ())

        def slab(i_local, k, u2, out_wait_pred, last_item, not_last=None):
            wait_gather(k % 4)
            if k >= 2 or out_wait_pred is None:
                wait_out(k % 2)
            else:
                @pl.when(out_wait_pred)
                def _():
                    wait_out(k % 2)
            transpose_rows(k % 4, k % 2)
            fire_out(i_local, k)
            if k <= SB - 3:
                fire_gather(k + 2, (k + 2) % 4, u2)
            if not last_item:
                if k == 6:
                    @pl.when(not_last)
                    def _():
                        wait_idx(1 - u2)
                if k == 7:
                    @pl.when(not_last)
                    def _():
                        fire_gather(0, 0, 1 - u2)
                        fire_gather(1, 1, 1 - u2)

        st0, bt0 = coords(0)
        pltpu.sync_copy(x_hbm.at[st0, bt0], idx_v.at[0])
        fire_idx(1, 1)
        fire_gather(0, 0, 0)
        fire_gather(1, 1, 0)

        def item_step(i_local, _):
            u = i_local % 2
            not_last = i_local < per_w - 1
            for k in range(SB):
                pred = (i_local > 0) if k < 2 else None
                slab(i_local, k, u, pred, last_item=False, not_last=not_last)
            @pl.when(i_local < per_w - 2)
            def _():
                fire_idx(i_local + 2, u)
            return ()

        lax.fori_loop(0, per_w, item_step, ())
        wait_out(0)
        wait_out(1)

    return body(x4, t2)


def kernel(x, table):
    b, s = x.shape
    n_sblk, n_bblk = s // SB, b // BB
    x4 = (
        x.T.astype(jnp.int32)
        .reshape(n_sblk, SB, n_bblk, BB)
        .transpose(0, 2, 1, 3)
    )
    scr = _format_table(table.T)
    t2 = scr.reshape(VOCAB_PAD, D_MODEL)
    o4 = _embed_lookup(x4, t2)
    o5 = o4.reshape(s, D_MODEL // 8, n_bblk, 8, BB)
    return o5.transpose(2, 4, 0, 1, 3).reshape(b, s, D_MODEL)


# speedup vs baseline: 1.2156x; 1.1576x over previous
import functools

import jax
import jax.numpy as jnp
from jax import lax
from jax.experimental import pallas as pl
from jax.experimental.pallas import tpu as pltpu
from jax.experimental.pallas import tpu_sc as plsc

D_MODEL = 64
SCALE = 8.0

NC = 2
NS = 16
NW = NC * NS
LANES = 16

SB = 8
BB = 128

VOCAB_PAD = 1000064
NBLK = VOCAB_PAD // 128
ABLK = 245


@jax.jit
def _format_table(table_t):
    mesh = plsc.VectorSubcoreMesh(
        core_axis_name="c", subcore_axis_name="s", num_cores=NC, num_subcores=NS
    )

    @functools.partial(
        pl.kernel,
        out_type=jax.ShapeDtypeStruct((VOCAB_PAD * D_MODEL,), jnp.float32),
        mesh=mesh,
        scratch_types=[
            pltpu.VMEM((2, D_MODEL, 128), jnp.float32),
            pltpu.VMEM((D_MODEL * 128,), jnp.float32),
            pltpu.VMEM((D_MODEL * 128,), jnp.float32),
            pltpu.SemaphoreType.DMA((2,)),
            pltpu.SemaphoreType.DMA((2,)),
        ],
        compiler_params=pltpu.CompilerParams(
            use_tc_tiling_on_sc=True,
            needs_layout_passes=False,
            disable_bounds_checks=True,
        ),
    )
    def body(tt_hbm, scr_hbm, stage, obuf_0, obuf_1, sem_i, sem_o):
        obufs = [obuf_0, obuf_1]
        wid = lax.axis_index("s") * NC + lax.axis_index("c")
        base = (NBLK * wid) // NW
        iota = lax.iota(jnp.int32, LANES)
        rot = [(iota + dd) & 15 for dd in range(LANES)]
        q64 = [iota * D_MODEL + rot[dd] for dd in range(LANES)]

        def start_in(i, slot):
            v0 = pl.multiple_of((base + i) * 128, 128)
            return pltpu.async_copy(
                tt_hbm.at[:, pl.ds(v0, 128)], stage.at[slot], sem_i.at[slot]
            )

        def start_out(i, slot):
            r0 = pl.multiple_of((base + i) * (128 * D_MODEL), 128 * D_MODEL)
            return pltpu.async_copy(
                obufs[slot],
                scr_hbm.at[pl.ds(r0, 128 * D_MODEL)],
                sem_o.at[slot],
            )

        def transpose_block(slot):
            def one_tile(t, _):
                jd = t // 2
                gq = (t % 2) * 4
                vvs = [iota + (gq + u) * 16 for u in range(4)]
                woffs = [(gq + u) * (16 * D_MODEL) + jd * 16 for u in range(4)]
                for dd in range(LANES):
                    dv = rot[dd] + jd * 16
                    vecs = [
                        plsc.load_gather(stage.at[slot], [dv, vvs[u]])
                        for u in range(4)
                    ]
                    for u in range(4):
                        plsc.store_scatter(
                            obufs[slot], [q64[dd] + woffs[u]], vecs[u] * SCALE
                        )
                return ()

            lax.fori_loop(0, 8, one_tile, ())

        start_in(0, 0)
        start_in(1, 1)

        def step(i2, _):
            for b in range(2):
                i = i2 * 2 + b
                pltpu.make_async_copy(
                    tt_hbm.at[:, pl.ds(0, 128)], stage.at[b], sem_i.at[b]
                ).wait()

                @pl.when(i2 >= 1)
                def _():
                    pltpu.make_async_copy(
                        obufs[b],
                        scr_hbm.at[pl.ds(0, 128 * D_MODEL)],
                        sem_o.at[b],
                    ).wait()

                transpose_block(b)
                start_out(i, b)
                if b == 0:
                    start_in(i + 2, 0)
                else:
                    @pl.when(i2 < (ABLK - 3) // 2)
                    def _():
                        start_in(i + 2, 1)
            return ()

        lax.fori_loop(0, (ABLK - 1) // 2, step, ())
        i_last = ABLK - 1
        pltpu.make_async_copy(
            tt_hbm.at[:, pl.ds(0, 128)], stage.at[0], sem_i.at[0]
        ).wait()
        pltpu.make_async_copy(
            obufs[0], scr_hbm.at[pl.ds(0, 128 * D_MODEL)], sem_o.at[0]
        ).wait()
        transpose_block(0)
        start_out(i_last, 0)
        pltpu.make_async_copy(
            obufs[1], scr_hbm.at[pl.ds(0, 128 * D_MODEL)], sem_o.at[1]
        ).wait()
        pltpu.make_async_copy(
            obufs[0], scr_hbm.at[pl.ds(0, 128 * D_MODEL)], sem_o.at[0]
        ).wait()

    return body(table_t)


@jax.jit
def _embed_lookup(x4, t2):
    n_sblk, n_bblk = x4.shape[0], x4.shape[1]
    per_w = (n_sblk * n_bblk) // NW
    mesh = plsc.VectorSubcoreMesh(
        core_axis_name="c", subcore_axis_name="s", num_cores=NC, num_subcores=NS
    )

    @functools.partial(
        pl.kernel,
        out_type=jax.ShapeDtypeStruct(
            (n_sblk * SB, D_MODEL // 8, n_bblk, 8 * BB), jnp.float32
        ),
        mesh=mesh,
        scratch_types=[
            pltpu.VMEM((2, SB, BB), jnp.int32),
            pltpu.VMEM((4, BB, D_MODEL), jnp.float32),
            pltpu.VMEM((D_MODEL * BB,), jnp.float32),
            pltpu.VMEM((D_MODEL * BB,), jnp.float32),
            pltpu.SemaphoreType.DMA((4,)),
            pltpu.SemaphoreType.DMA((2,)),
            pltpu.SemaphoreType.DMA((2,)),
        ],
        compiler_params=pltpu.CompilerParams(
            use_tc_tiling_on_sc=False, needs_layout_passes=False
        ),
    )
    def body(
        x_hbm, t_hbm, out_hbm, idx_v, rows_v, obuf_0, obuf_1,
        sem_g, sem_o, sem_x,
    ):
        obufs = [obuf_0, obuf_1]
        wid = lax.axis_index("s") * NC + lax.axis_index("c")
        iota = lax.iota(jnp.int32, LANES)
        rot = [(iota + dd) & 15 for dd in range(LANES)]
        p128 = [rot[dd] * BB + iota for dd in range(LANES)]

        def coords(i_local):
            it = wid * per_w + i_local
            return it // n_bblk, it % n_bblk

        def fire_idx(i_local, xs):
            st, bt = coords(i_local)
            pltpu.async_copy(x_hbm.at[st, bt], idx_v.at[xs], sem_x.at[xs])

        def wait_idx(xs):
            pltpu.make_async_copy(
                x_hbm.at[0, 0], idx_v.at[xs], sem_x.at[xs]
            ).wait()

        def fire_gather(k, gs, xs):
            pltpu.async_copy(
                t_hbm.at[idx_v.at[xs, k]], rows_v.at[gs], sem_g.at[gs]
            )

        def wait_gather(gs):
            pltpu.make_async_copy(
                t_hbm.at[idx_v.at[0, 0]], rows_v.at[gs], sem_g.at[gs]
            ).wait()

        def fire_out(i_local, k):
            st, bt = coords(i_local)
            s = st * SB + k
            for dt in range(D_MODEL // 8):
                pltpu.async_copy(
                    obufs[k % 2].at[pl.ds(dt * (8 * BB), 8 * BB)],
                    out_hbm.at[s, dt, bt],
                    sem_o.at[k % 2],
                )

        def wait_out(oslot):
            for dt in range(D_MODEL // 8):
                pltpu.make_async_copy(
                    obufs[oslot].at[pl.ds(dt * (8 * BB), 8 * BB)],
                    out_hbm.at[0, 0, 0],
                    sem_o.at[oslot],
                ).wait()

        def transpose_rows(jr, jo):
            def one_tile(g4, _):
                gbs = [g4 * 2, g4 * 2 + 1]
                bbs = [iota + gb * 16 for gb in gbs]
                woffs = [
                    [jd * (16 * BB) + gb * 16 for jd in range(4)]
                    for gb in gbs
                ]
                for dd in range(LANES):
                    vecs = [
                        plsc.load_gather(
                            rows_v.at[jr], [bbs[t], rot[dd] + jd * 16]
                        )
                        for t in range(2)
                        for jd in range(4)
                    ]
                    for t in range(2):
                        for jd in range(4):
                            plsc.store_scatter(
                                obufs[jo],
                                [p128[dd] + woffs[t][jd]],
                                vecs[t * 4 + jd],
                            )
                return ()

            lax.fori_loop(0, 4, one_tile, ())

        def slab(i_local, k, u2, out_wait_pred, last_item, not_last=None):
            wait_gather(k % 4)
            if k >= 2 or out_wait_pred is None:
                wait_out(k % 2)
            else:
                @pl.when(out_wait_pred)
                def _():
                    wait_out(k % 2)
            transpose_rows(k % 4, k % 2)
            fire_out(i_local, k)
            if k <= SB - 3:
                fire_gather(k + 2, (k + 2) % 4, u2)
            if not last_item:
                if k == 6:
                    @pl.when(not_last)
                    def _():
                        wait_idx(1 - u2)
                if k == 7:
                    @pl.when(not_last)
                    def _():
                        fire_gather(0, 0, 1 - u2)
                        fire_gather(1, 1, 1 - u2)

        st0, bt0 = coords(0)
        pltpu.sync_copy(x_hbm.at[st0, bt0], idx_v.at[0])
        fire_idx(1, 1)
        fire_gather(0, 0, 0)
        fire_gather(1, 1, 0)

        def item_step(i_local, _):
            u = i_local % 2
            not_last = i_local < per_w - 1
            for k in range(SB):
                pred = (i_local > 0) if k < 2 else None
                slab(i_local, k, u, pred, last_item=False, not_last=not_last)
            @pl.when(i_local < per_w - 2)
            def _():
                fire_idx(i_local + 2, u)
            return ()

        lax.fori_loop(0, per_w, item_step, ())
        wait_out(0)
        wait_out(1)

    return body(x4, t2)


def kernel(x, table):
    b, s = x.shape
    n_sblk, n_bblk = s // SB, b // BB
    x4 = (
        x.T.astype(jnp.int32)
        .reshape(n_sblk, SB, n_bblk, BB)
        .transpose(0, 2, 1, 3)
    )
    scr = _format_table(table.T)
    t2 = scr.reshape(VOCAB_PAD, D_MODEL)
    o4 = _embed_lookup(x4, t2)
    o5 = o4.reshape(s, D_MODEL // 8, n_bblk, 8, BB)
    return o5.transpose(2, 4, 0, 1, 3).reshape(b, s, D_MODEL)


# speedup vs baseline: 1.2932x; 1.0638x over previous
import functools

import jax
import jax.numpy as jnp
from jax import lax
from jax.experimental import pallas as pl
from jax.experimental.pallas import tpu as pltpu
from jax.experimental.pallas import tpu_sc as plsc

D_MODEL = 64
SCALE = 8.0

NC = 2
NS = 16
NW = NC * NS
LANES = 16

SB = 8
BB = 128

VOCAB_PAD = 1000064
NBLK = VOCAB_PAD // 128
ABLK = 245


@jax.jit
def _format_table(table_t):
    mesh = plsc.VectorSubcoreMesh(
        core_axis_name="c", subcore_axis_name="s", num_cores=NC, num_subcores=NS
    )

    @functools.partial(
        pl.kernel,
        out_type=jax.ShapeDtypeStruct((VOCAB_PAD * D_MODEL,), jnp.float32),
        mesh=mesh,
        scratch_types=[
            pltpu.VMEM((2, D_MODEL, 128), jnp.float32),
            pltpu.VMEM((D_MODEL * 128,), jnp.float32),
            pltpu.VMEM((D_MODEL * 128,), jnp.float32),
            pltpu.SemaphoreType.DMA((2,)),
            pltpu.SemaphoreType.DMA((2,)),
        ],
        compiler_params=pltpu.CompilerParams(
            use_tc_tiling_on_sc=True,
            needs_layout_passes=False,
            disable_bounds_checks=True,
        ),
    )
    def body(tt_hbm, scr_hbm, stage, obuf_0, obuf_1, sem_i, sem_o):
        obufs = [obuf_0, obuf_1]
        wid = lax.axis_index("s") * NC + lax.axis_index("c")
        base = (NBLK * wid) // NW
        iota = lax.iota(jnp.int32, LANES)
        rot = [(iota + dd) & 15 for dd in range(LANES)]
        q64 = [iota * D_MODEL + rot[dd] for dd in range(LANES)]

        def start_in(i, slot):
            v0 = pl.multiple_of((base + i) * 128, 128)
            return pltpu.async_copy(
                tt_hbm.at[:, pl.ds(v0, 128)], stage.at[slot], sem_i.at[slot]
            )

        def start_out(i, slot):
            r0 = pl.multiple_of((base + i) * (128 * D_MODEL), 128 * D_MODEL)
            return pltpu.async_copy(
                obufs[slot],
                scr_hbm.at[pl.ds(r0, 128 * D_MODEL)],
                sem_o.at[slot],
            )

        def transpose_block(slot):
            def one_tile(jd, _):
                vvs = [iota + u * 16 for u in range(8)]
                woffs = [u * (16 * D_MODEL) + jd * 16 for u in range(8)]
                for dd in range(LANES):
                    dv = rot[dd] + jd * 16
                    vecs = [
                        plsc.load_gather(stage.at[slot], [dv, vvs[u]])
                        for u in range(8)
                    ]
                    for u in range(8):
                        plsc.store_scatter(
                            obufs[slot], [q64[dd] + woffs[u]], vecs[u] * SCALE
                        )
                return ()

            lax.fori_loop(0, 4, one_tile, ())

        start_in(0, 0)
        start_in(1, 1)

        def step(i2, _):
            for b in range(2):
                i = i2 * 2 + b
                pltpu.make_async_copy(
                    tt_hbm.at[:, pl.ds(0, 128)], stage.at[b], sem_i.at[b]
                ).wait()

                @pl.when(i2 >= 1)
                def _():
                    pltpu.make_async_copy(
                        obufs[b],
                        scr_hbm.at[pl.ds(0, 128 * D_MODEL)],
                        sem_o.at[b],
                    ).wait()

                transpose_block(b)
                start_out(i, b)
                if b == 0:
                    start_in(i + 2, 0)
                else:
                    @pl.when(i2 < (ABLK - 3) // 2)
                    def _():
                        start_in(i + 2, 1)
            return ()

        lax.fori_loop(0, (ABLK - 1) // 2, step, ())
        i_last = ABLK - 1
        pltpu.make_async_copy(
            tt_hbm.at[:, pl.ds(0, 128)], stage.at[0], sem_i.at[0]
        ).wait()
        pltpu.make_async_copy(
            obufs[0], scr_hbm.at[pl.ds(0, 128 * D_MODEL)], sem_o.at[0]
        ).wait()
        transpose_block(0)
        start_out(i_last, 0)
        pltpu.make_async_copy(
            obufs[1], scr_hbm.at[pl.ds(0, 128 * D_MODEL)], sem_o.at[1]
        ).wait()
        pltpu.make_async_copy(
            obufs[0], scr_hbm.at[pl.ds(0, 128 * D_MODEL)], sem_o.at[0]
        ).wait()

    return body(table_t)


@jax.jit
def _embed_lookup(x4, t2):
    n_sblk, n_bblk = x4.shape[0], x4.shape[1]
    per_w = (n_sblk * n_bblk) // NW
    mesh = plsc.VectorSubcoreMesh(
        core_axis_name="c", subcore_axis_name="s", num_cores=NC, num_subcores=NS
    )

    @functools.partial(
        pl.kernel,
        out_type=jax.ShapeDtypeStruct(
            (n_sblk * SB, D_MODEL // 8, n_bblk, 8 * BB), jnp.float32
        ),
        mesh=mesh,
        scratch_types=[
            pltpu.VMEM((2, SB, BB), jnp.int32),
            pltpu.VMEM((4, BB, D_MODEL), jnp.float32),
            pltpu.VMEM((D_MODEL * BB,), jnp.float32),
            pltpu.VMEM((D_MODEL * BB,), jnp.float32),
            pltpu.SemaphoreType.DMA((4,)),
            pltpu.SemaphoreType.DMA((2,)),
            pltpu.SemaphoreType.DMA((2,)),
        ],
        compiler_params=pltpu.CompilerParams(
            use_tc_tiling_on_sc=False, needs_layout_passes=False
        ),
    )
    def body(
        x_hbm, t_hbm, out_hbm, idx_v, rows_v, obuf_0, obuf_1,
        sem_g, sem_o, sem_x,
    ):
        obufs = [obuf_0, obuf_1]
        wid = lax.axis_index("s") * NC + lax.axis_index("c")
        iota = lax.iota(jnp.int32, LANES)
        rot = [(iota + dd) & 15 for dd in range(LANES)]
        p128 = [rot[dd] * BB + iota for dd in range(LANES)]

        def coords(i_local):
            it = wid * per_w + i_local
            return it // n_bblk, it % n_bblk

        def fire_idx(i_local, xs):
            st, bt = coords(i_local)
            pltpu.async_copy(x_hbm.at[st, bt], idx_v.at[xs], sem_x.at[xs])

        def wait_idx(xs):
            pltpu.make_async_copy(
                x_hbm.at[0, 0], idx_v.at[xs], sem_x.at[xs]
            ).wait()

        def fire_gather(k, gs, xs):
            pltpu.async_copy(
                t_hbm.at[idx_v.at[xs, k]], rows_v.at[gs], sem_g.at[gs]
            )

        def wait_gather(gs):
            pltpu.make_async_copy(
                t_hbm.at[idx_v.at[0, 0]], rows_v.at[gs], sem_g.at[gs]
            ).wait()

        def fire_out(i_local, k):
            st, bt = coords(i_local)
            s = st * SB + k
            for dt in range(D_MODEL // 8):
                pltpu.async_copy(
                    obufs[k % 2].at[pl.ds(dt * (8 * BB), 8 * BB)],
                    out_hbm.at[s, dt, bt],
                    sem_o.at[k % 2],
                )

        def wait_out(oslot):
            for dt in range(D_MODEL // 8):
                pltpu.make_async_copy(
                    obufs[oslot].at[pl.ds(dt * (8 * BB), 8 * BB)],
                    out_hbm.at[0, 0, 0],
                    sem_o.at[oslot],
                ).wait()

        def transpose_rows(jr, jo):
            def one_tile(g4, _):
                gbs = [g4 * 2, g4 * 2 + 1]
                bbs = [iota + gb * 16 for gb in gbs]
                woffs = [
                    [jd * (16 * BB) + gb * 16 for jd in range(4)]
                    for gb in gbs
                ]
                for dd in range(LANES):
                    vecs = [
                        plsc.load_gather(
                            rows_v.at[jr], [bbs[t], rot[dd] + jd * 16]
                        )
                        for t in range(2)
                        for jd in range(4)
                    ]
                    for t in range(2):
                        for jd in range(4):
                            plsc.store_scatter(
                                obufs[jo],
                                [p128[dd] + woffs[t][jd]],
                                vecs[t * 4 + jd],
                            )
                return ()

            lax.fori_loop(0, 4, one_tile, ())

        def slab(i_local, k, u2, out_wait_pred, last_item, not_last=None):
            wait_gather(k % 4)
            if k >= 2 or out_wait_pred is None:
                wait_out(k % 2)
            else:
                @pl.when(out_wait_pred)
                def _():
                    wait_out(k % 2)
            transpose_rows(k % 4, k % 2)
            fire_out(i_local, k)
            if k <= SB - 3:
                fire_gather(k + 2, (k + 2) % 4, u2)
            if not last_item:
                if k == 6:
                    @pl.when(not_last)
                    def _():
                        wait_idx(1 - u2)
                if k == 7:
                    @pl.when(not_last)
                    def _():
                        fire_gather(0, 0, 1 - u2)
                        fire_gather(1, 1, 1 - u2)

        st0, bt0 = coords(0)
        pltpu.sync_copy(x_hbm.at[st0, bt0], idx_v.at[0])
        fire_idx(1, 1)
        fire_gather(0, 0, 0)
        fire_gather(1, 1, 0)

        def item_step(i_local, _):
            u = i_local % 2
            not_last = i_local < per_w - 1
            for k in range(SB):
                pred = (i_local > 0) if k < 2 else None
                slab(i_local, k, u, pred, last_item=False, not_last=not_last)
            @pl.when(i_local < per_w - 2)
            def _():
                fire_idx(i_local + 2, u)
            return ()

        lax.fori_loop(0, per_w, item_step, ())
        wait_out(0)
        wait_out(1)

    return body(x4, t2)


def kernel(x, table):
    b, s = x.shape
    n_sblk, n_bblk = s // SB, b // BB
    x4 = (
        x.T.astype(jnp.int32)
        .reshape(n_sblk, SB, n_bblk, BB)
        .transpose(0, 2, 1, 3)
    )
    scr = _format_table(table.T)
    t2 = scr.reshape(VOCAB_PAD, D_MODEL)
    o4 = _embed_lookup(x4, t2)
    o5 = o4.reshape(s, D_MODEL // 8, n_bblk, 8, BB)
    return o5.transpose(2, 4, 0, 1, 3).reshape(b, s, D_MODEL)


# speedup vs baseline: 1.3538x; 1.0468x over previous
import functools

import jax
import jax.numpy as jnp
from jax import lax
from jax.experimental import pallas as pl
from jax.experimental.pallas import tpu as pltpu
from jax.experimental.pallas import tpu_sc as plsc

D_MODEL = 64
SCALE = 8.0

NC = 2
NS = 16
NW = NC * NS
LANES = 16

SB = 8
BB = 128

VOCAB_PAD = 1000064
NBLK = VOCAB_PAD // 128
ABLK = 245


@jax.jit
def _format_table(table_t):
    mesh = plsc.VectorSubcoreMesh(
        core_axis_name="c", subcore_axis_name="s", num_cores=NC, num_subcores=NS
    )

    @functools.partial(
        pl.kernel,
        out_type=jax.ShapeDtypeStruct((VOCAB_PAD * D_MODEL,), jnp.float32),
        mesh=mesh,
        scratch_types=[
            pltpu.VMEM((2, D_MODEL, 128), jnp.float32),
            pltpu.VMEM((D_MODEL * 128,), jnp.float32),
            pltpu.VMEM((D_MODEL * 128,), jnp.float32),
            pltpu.SemaphoreType.DMA((2,)),
            pltpu.SemaphoreType.DMA((2,)),
        ],
        compiler_params=pltpu.CompilerParams(
            use_tc_tiling_on_sc=True,
            needs_layout_passes=False,
            disable_bounds_checks=True,
        ),
    )
    def body(tt_hbm, scr_hbm, stage, obuf_0, obuf_1, sem_i, sem_o):
        obufs = [obuf_0, obuf_1]
        wid = lax.axis_index("s") * NC + lax.axis_index("c")
        base = (NBLK * wid) // NW
        iota = lax.iota(jnp.int32, LANES)
        rot = [(iota + dd) & 15 for dd in range(LANES)]
        q64 = [iota * D_MODEL + rot[dd] for dd in range(LANES)]

        def start_in(i, slot):
            v0 = pl.multiple_of((base + i) * 128, 128)
            return pltpu.async_copy(
                tt_hbm.at[:, pl.ds(v0, 128)], stage.at[slot], sem_i.at[slot]
            )

        def start_out(i, slot):
            r0 = pl.multiple_of((base + i) * (128 * D_MODEL), 128 * D_MODEL)
            return pltpu.async_copy(
                obufs[slot],
                scr_hbm.at[pl.ds(r0, 128 * D_MODEL)],
                sem_o.at[slot],
            )

        def transpose_block(slot):
            def one_tile(jd, _):
                vvs = [iota + u * 16 for u in range(8)]
                woffs = [u * (16 * D_MODEL) + jd * 16 for u in range(8)]
                for dd in range(LANES):
                    dv = rot[dd] + jd * 16
                    vecs = [
                        plsc.load_gather(stage.at[slot], [dv, vvs[u]])
                        for u in range(8)
                    ]
                    for u in range(8):
                        plsc.store_scatter(
                            obufs[slot], [q64[dd] + woffs[u]], vecs[u] * SCALE
                        )
                return ()

            lax.fori_loop(0, 4, one_tile, ())

        start_in(0, 0)
        start_in(1, 1)

        def step(i2, _):
            for b in range(2):
                i = i2 * 2 + b
                pltpu.make_async_copy(
                    tt_hbm.at[:, pl.ds(0, 128)], stage.at[b], sem_i.at[b]
                ).wait()

                @pl.when(i2 >= 1)
                def _():
                    pltpu.make_async_copy(
                        obufs[b],
                        scr_hbm.at[pl.ds(0, 128 * D_MODEL)],
                        sem_o.at[b],
                    ).wait()

                transpose_block(b)
                start_out(i, b)
                if b == 0:
                    start_in(i + 2, 0)
                else:
                    @pl.when(i2 < (ABLK - 3) // 2)
                    def _():
                        start_in(i + 2, 1)
            return ()

        lax.fori_loop(0, (ABLK - 1) // 2, step, ())
        i_last = ABLK - 1
        pltpu.make_async_copy(
            tt_hbm.at[:, pl.ds(0, 128)], stage.at[0], sem_i.at[0]
        ).wait()
        pltpu.make_async_copy(
            obufs[0], scr_hbm.at[pl.ds(0, 128 * D_MODEL)], sem_o.at[0]
        ).wait()
        transpose_block(0)
        start_out(i_last, 0)
        pltpu.make_async_copy(
            obufs[1], scr_hbm.at[pl.ds(0, 128 * D_MODEL)], sem_o.at[1]
        ).wait()
        pltpu.make_async_copy(
            obufs[0], scr_hbm.at[pl.ds(0, 128 * D_MODEL)], sem_o.at[0]
        ).wait()

    return body(table_t)


@jax.jit
def _embed_lookup(x4, t2):
    n_sblk, n_bblk = x4.shape[0], x4.shape[1]
    per_w = (n_sblk * n_bblk) // NW
    mesh = plsc.VectorSubcoreMesh(
        core_axis_name="c", subcore_axis_name="s", num_cores=NC, num_subcores=NS
    )

    @functools.partial(
        pl.kernel,
        out_type=jax.ShapeDtypeStruct(
            (n_sblk * SB, D_MODEL // 8, n_bblk, 8 * BB), jnp.float32
        ),
        mesh=mesh,
        scratch_types=[
            pltpu.VMEM((2, SB, BB), jnp.int32),
            pltpu.VMEM((4, BB, D_MODEL), jnp.float32),
            pltpu.VMEM((D_MODEL * BB,), jnp.float32),
            pltpu.VMEM((D_MODEL * BB,), jnp.float32),
            pltpu.SemaphoreType.DMA((4,)),
            pltpu.SemaphoreType.DMA((2,)),
            pltpu.SemaphoreType.DMA((2,)),
        ],
        compiler_params=pltpu.CompilerParams(
            use_tc_tiling_on_sc=False, needs_layout_passes=False
        ),
    )
    def body(
        x_hbm, t_hbm, out_hbm, idx_v, rows_v, obuf_0, obuf_1,
        sem_g, sem_o, sem_x,
    ):
        obufs = [obuf_0, obuf_1]
        wid = lax.axis_index("s") * NC + lax.axis_index("c")
        iota = lax.iota(jnp.int32, LANES)
        rot = [(iota + dd) & 15 for dd in range(LANES)]
        p128 = [rot[dd] * BB + iota for dd in range(LANES)]

        def coords(i_local):
            it = wid * per_w + i_local
            return it // n_bblk, it % n_bblk

        def fire_idx(i_local, xs):
            st, bt = coords(i_local)
            pltpu.async_copy(x_hbm.at[st, bt], idx_v.at[xs], sem_x.at[xs])

        def wait_idx(xs):
            pltpu.make_async_copy(
                x_hbm.at[0, 0], idx_v.at[xs], sem_x.at[xs]
            ).wait()

        def fire_gather(k, gs, xs):
            pltpu.async_copy(
                t_hbm.at[idx_v.at[xs, k]], rows_v.at[gs], sem_g.at[gs]
            )

        def wait_gather(gs):
            pltpu.make_async_copy(
                t_hbm.at[idx_v.at[0, 0]], rows_v.at[gs], sem_g.at[gs]
            ).wait()

        def fire_out(i_local, k):
            st, bt = coords(i_local)
            s = st * SB + k
            for dt in range(D_MODEL // 8):
                pltpu.async_copy(
                    obufs[k % 2].at[pl.ds(dt * (8 * BB), 8 * BB)],
                    out_hbm.at[s, dt, bt],
                    sem_o.at[k % 2],
                )

        def wait_out(oslot):
            for dt in range(D_MODEL // 8):
                pltpu.make_async_copy(
                    obufs[oslot].at[pl.ds(dt * (8 * BB), 8 * BB)],
                    out_hbm.at[0, 0, 0],
                    sem_o.at[oslot],
                ).wait()

        def transpose_rows(jr, jo):
            def one_tile(g4, _):
                gbs = [g4 * 2, g4 * 2 + 1]
                bbs = [iota + gb * 16 for gb in gbs]
                woffs = [
                    [jd * (16 * BB) + gb * 16 for jd in range(4)]
                    for gb in gbs
                ]
                for dd in range(LANES):
                    vecs = [
                        plsc.load_gather(
                            rows_v.at[jr], [bbs[t], rot[dd] + jd * 16]
                        )
                        for t in range(2)
                        for jd in range(4)
                    ]
                    for t in range(2):
                        for jd in range(4):
                            plsc.store_scatter(
                                obufs[jo],
                                [p128[dd] + woffs[t][jd]],
                                vecs[t * 4 + jd],
                            )
                return ()

            lax.fori_loop(0, 4, one_tile, ())

        def slab(i_local, k, u2, out_wait_pred, last_item, not_last=None):
            wait_gather(k % 4)
            if k >= 2 or out_wait_pred is None:
                wait_out(k % 2)
            else:
                @pl.when(out_wait_pred)
                def _():
                    wait_out(k % 2)
            transpose_rows(k % 4, k % 2)
            fire_out(i_local, k)
            if k <= SB - 4:
                fire_gather(k + 3, (k + 3) % 4, u2)
            if not last_item:
                if k == 6:
                    @pl.when(not_last)
                    def _():
                        wait_idx(1 - u2)
                if k == 7:
                    @pl.when(not_last)
                    def _():
                        fire_gather(0, 0, 1 - u2)
                        fire_gather(1, 1, 1 - u2)
                        fire_gather(2, 2, 1 - u2)

        st0, bt0 = coords(0)
        pltpu.sync_copy(x_hbm.at[st0, bt0], idx_v.at[0])
        fire_idx(1, 1)
        fire_gather(0, 0, 0)
        fire_gather(1, 1, 0)
        fire_gather(2, 2, 0)

        def item_step(i_local, _):
            u = i_local % 2
            not_last = i_local < per_w - 1
            for k in range(SB):
                pred = (i_local > 0) if k < 2 else None
                slab(i_local, k, u, pred, last_item=False, not_last=not_last)
            @pl.when(i_local < per_w - 2)
            def _():
                fire_idx(i_local + 2, u)
            return ()

        lax.fori_loop(0, per_w, item_step, ())
        wait_out(0)
        wait_out(1)

    return body(x4, t2)


def kernel(x, table):
    b, s = x.shape
    n_sblk, n_bblk = s // SB, b // BB
    x4 = (
        x.T.astype(jnp.int32)
        .reshape(n_sblk, SB, n_bblk, BB)
        .transpose(0, 2, 1, 3)
    )
    scr = _format_table(table.T)
    t2 = scr.reshape(VOCAB_PAD, D_MODEL)
    o4 = _embed_lookup(x4, t2)
    o5 = o4.reshape(s, D_MODEL // 8, n_bblk, 8, BB)
    return o5.transpose(2, 4, 0, 1, 3).reshape(b, s, D_MODEL)
